# baseline (speedup 1.0000x reference)
import functools

import jax
import jax.numpy as jnp
from jax import lax
from jax.experimental import pallas as pl
from jax.experimental.pallas import tpu as pltpu
from jax.experimental.pallas import tpu_sc as plsc

_K = 100000
_D = 128
_B = 16384
_NC = 2
_NS = 16
_NW = _NC * _NS
_ROWS = _B // _NW
_G = 16
_NGRP = _ROWS // _G


def _xbm_body(params_hbm, feats_hbm, tcat_hbm, fmem_hbm, tmcat_hbm,
              outf_hbm, outt_hbm,
              params_v, fbuf, tlobuf, thibuf, mstage, tstage,
              sem, sem2, sem3, sem4, sem5):
    wid = lax.axis_index("s") * _NC + lax.axis_index("c")
    base = wid * _ROWS
    dst = pl.multiple_of(base, _ROWS)
    csz = _ROWS // 4
    fsems = (sem, sem2, sem3, sem4)
    dsts = [pl.multiple_of(base + i * csz, csz) for i in range(4)]

    ain = [pltpu.async_copy(feats_hbm.at[pl.ds(dsts[i], csz)],
                            fbuf.at[pl.ds(i * csz, csz)], fsems[i])
           for i in range(4)]
    t0 = pltpu.async_copy(tcat_hbm.at[pl.ds(dst, _ROWS)], tlobuf, sem5)
    t1 = pltpu.async_copy(tcat_hbm.at[pl.ds(_B + dst, _ROWS)], thibuf, sem5)

    pltpu.sync_copy(params_hbm, params_v)
    pv = params_v[...]
    ws = pv[0]
    os_ = pv[1]
    g0 = os_ + base

    spec_ok = ws == os_

    full_f = jnp.logical_and(g0 >= ws, g0 + _ROWS <= ws + _B)
    full_m = jnp.logical_or(g0 + _ROWS <= ws, g0 >= ws + _B)
    src_f = g0 - ws
    case_a = jnp.logical_and(
        jnp.logical_and(full_f, src_f % _ROWS == 0),
        jnp.logical_not(spec_ok))
    case_b = jnp.logical_and(full_m, g0 % _ROWS == 0)
    case_c = jnp.logical_not(jnp.logical_or(
        jnp.logical_or(case_a, case_b), spec_ok))

    @pl.when(spec_ok)
    def _():
        outs = []
        for i in range(4):
            ain[i].wait()
            outs.append(pltpu.async_copy(
                fbuf.at[pl.ds(i * csz, csz)],
                outf_hbm.at[pl.ds(dsts[i], csz)], fsems[i]))
        t0.wait()
        t1.wait()
        outs.append(pltpu.async_copy(
            tlobuf, outt_hbm.at[pl.ds(dst, _ROWS)], sem5))
        outs.append(pltpu.async_copy(
            thibuf, outt_hbm.at[pl.ds(_B + dst, _ROWS)], sem5))
        for o in outs:
            o.wait()

    @pl.when(jnp.logical_not(spec_ok))
    def _():
        for a in ain:
            a.wait()
        t0.wait()
        t1.wait()

    @pl.when(case_a)
    def _():
        src = pl.multiple_of(src_f, _ROWS)
        pltpu.sync_copy(feats_hbm.at[pl.ds(src, _ROWS)], fbuf)
        pltpu.sync_copy(tcat_hbm.at[pl.ds(src, _ROWS)], tlobuf)
        pltpu.sync_copy(tcat_hbm.at[pl.ds(_B + src, _ROWS)], thibuf)

    @pl.when(case_b)
    def _():
        src = pl.multiple_of(g0, _ROWS)
        pltpu.sync_copy(fmem_hbm.at[pl.ds(src, _ROWS)], fbuf)
        pltpu.sync_copy(tmcat_hbm.at[pl.ds(src, _ROWS)], tlobuf)
        pltpu.sync_copy(tmcat_hbm.at[pl.ds(_K + src, _ROWS)], thibuf)

    @pl.when(case_c)
    def _():
        iota = lax.iota(jnp.int32, 16)

        def group(gi, carry):
            off = gi * _G
            c0 = g0 + off
            gvec = c0 + iota
            validv = jnp.logical_and(gvec >= ws, gvec < ws + _B)
            fidx = jnp.clip(gvec - ws, 0, _B - 1)

            pltpu.async_copy(feats_hbm.at[fidx],
                             fbuf.at[pl.ds(off, _G)], sem).wait()
            pltpu.async_copy(fmem_hbm.at[gvec], mstage, sem).wait()

            def fixrow(r, c2):
                g = c0 + r
                valid = jnp.logical_and(g >= ws, g < ws + _B)

                @pl.when(jnp.logical_not(valid))
                def _():
                    for jc in range(_D // 16):
                        fbuf[off + r, pl.ds(jc * 16, 16)] = (
                            mstage[r, pl.ds(jc * 16, 16)])

                return c2

            lax.fori_loop(jnp.int32(0), jnp.int32(_G), fixrow, jnp.int32(0))

            for pbase, mbase, pbuf in ((0, 0, tlobuf), (_B, _K, thibuf)):
                pltpu.async_copy(tcat_hbm.at[pbase + fidx],
                                 pbuf.at[pl.ds(off, _G)], sem).wait()
                pltpu.async_copy(tmcat_hbm.at[mbase + gvec], tstage,
                                 sem).wait()
                pbuf[pl.ds(off, _G)] = jnp.where(
                    validv, pbuf[pl.ds(off, _G)], tstage[...])

            return carry

        lax.fori_loop(jnp.int32(0), jnp.int32(_NGRP), group, jnp.int32(0))

    @pl.when(jnp.logical_not(spec_ok))
    def _():
        pltpu.sync_copy(fbuf, outf_hbm.at[pl.ds(dst, _ROWS)])
        pltpu.sync_copy(tlobuf, outt_hbm.at[pl.ds(dst, _ROWS)])
        pltpu.sync_copy(thibuf, outt_hbm.at[pl.ds(_B + dst, _ROWS)])


_xbm_call = functools.partial(
    pl.kernel,
    out_type=[
        jax.ShapeDtypeStruct((_B, _D), jnp.float32),
        jax.ShapeDtypeStruct((2 * _B,), jnp.int32),
    ],
    mesh=plsc.VectorSubcoreMesh(core_axis_name="c", subcore_axis_name="s"),
    compiler_params=pltpu.CompilerParams(needs_layout_passes=False),
    scratch_types=[
        pltpu.VMEM((16,), jnp.int32),
        pltpu.VMEM((_ROWS, _D), jnp.float32),
        pltpu.VMEM((_ROWS,), jnp.int32),
        pltpu.VMEM((_ROWS,), jnp.int32),
        pltpu.VMEM((_G, _D), jnp.float32),
        pltpu.VMEM((_G,), jnp.int32),
        pltpu.SemaphoreType.DMA,
        pltpu.SemaphoreType.DMA,
        pltpu.SemaphoreType.DMA,
        pltpu.SemaphoreType.DMA,
        pltpu.SemaphoreType.DMA,
    ],
)(_xbm_body)


def kernel(feats, targets, feats_mem, targets_mem, ptr, total_count):
    q = feats.shape[0]
    ptr32 = jnp.clip(ptr, 0, _K + 1).astype(jnp.int32)
    tc32 = jnp.clip(total_count, 0, _K + 200).astype(jnp.int32)
    wrap = ptr32 + q > _K
    write_start = jnp.where(wrap, _K - q, ptr32)
    write_start = jnp.clip(write_start, 0, _K - q)
    new_ptr = jnp.where(wrap, 0, ptr32 + q)
    is_full = tc32 + q >= _K
    out_start = jnp.where(is_full, 0, new_ptr - q)
    out_start = jnp.clip(out_start, 0, _K - q)

    params = jnp.stack([write_start, out_start]).astype(jnp.int32)
    params = jnp.pad(params, (0, 14))

    t64 = targets.reshape(q)
    t_cat = jnp.concatenate(
        [t64.astype(jnp.int32), (t64 >> 32).astype(jnp.int32)])

    def _tmem_slow(tm):
        tm = lax.optimization_barrier(tm).reshape(_K)
        return jnp.concatenate(
            [tm.astype(jnp.int32), (tm >> 32).astype(jnp.int32)])

    tm_cat = lax.cond(
        write_start == out_start,
        lambda tm: jnp.zeros((2 * _K,), jnp.int32),
        _tmem_slow,
        targets_mem)

    out_feats, out_cat = _xbm_call(params, feats, t_cat, feats_mem, tm_cat)
    out_targets = (
        (out_cat[_B:].astype(jnp.int64) << 32)
        | (out_cat[:_B].astype(jnp.int64) & jnp.int64(0xFFFFFFFF))
    ).reshape(q, 1)
    return (out_feats, out_targets)


# speedup vs baseline: 1.0627x; 1.0627x over previous
import functools

import jax
import jax.numpy as jnp
from jax import lax
from jax.experimental import pallas as pl
from jax.experimental.pallas import tpu as pltpu
from jax.experimental.pallas import tpu_sc as plsc

_K = 100000
_D = 128
_B = 16384
_NC = 2
_NS = 16
_NW = _NC * _NS
_ROWS = _B // _NW
_G = 16
_NGRP = _ROWS // _G


def _xbm_body(params_hbm, feats_hbm, tcat_hbm, fmem_hbm, tmcat_hbm,
              outf_hbm, outt_hbm,
              params_v, fbuf, tlobuf, thibuf, mstage, tstage,
              sem, sem2, sem3):
    wid = lax.axis_index("s") * _NC + lax.axis_index("c")
    base = wid * _ROWS
    dst = pl.multiple_of(base, _ROWS)
    half = _ROWS // 2
    dst1 = pl.multiple_of(base + half, half)

    a0 = pltpu.async_copy(feats_hbm.at[pl.ds(dst, half)],
                          fbuf.at[pl.ds(0, half)], sem)
    a1 = pltpu.async_copy(feats_hbm.at[pl.ds(dst1, half)],
                          fbuf.at[pl.ds(half, half)], sem2)
    t0 = pltpu.async_copy(tcat_hbm.at[pl.ds(dst, _ROWS)], tlobuf, sem3)
    t1 = pltpu.async_copy(tcat_hbm.at[pl.ds(_B + dst, _ROWS)], thibuf, sem3)

    pltpu.sync_copy(params_hbm, params_v)
    pv = params_v[...]
    ws = pv[0]
    os_ = pv[1]
    g0 = os_ + base

    spec_ok = ws == os_

    full_f = jnp.logical_and(g0 >= ws, g0 + _ROWS <= ws + _B)
    full_m = jnp.logical_or(g0 + _ROWS <= ws, g0 >= ws + _B)
    src_f = g0 - ws
    case_a = jnp.logical_and(
        jnp.logical_and(full_f, src_f % _ROWS == 0),
        jnp.logical_not(spec_ok))
    case_b = jnp.logical_and(full_m, g0 % _ROWS == 0)
    case_c = jnp.logical_not(jnp.logical_or(
        jnp.logical_or(case_a, case_b), spec_ok))

    @pl.when(spec_ok)
    def _():
        a0.wait()
        o0 = pltpu.async_copy(fbuf.at[pl.ds(0, half)],
                              outf_hbm.at[pl.ds(dst, half)], sem)
        a1.wait()
        o1 = pltpu.async_copy(fbuf.at[pl.ds(half, half)],
                              outf_hbm.at[pl.ds(dst1, half)], sem2)
        t0.wait()
        t1.wait()
        ot0 = pltpu.async_copy(tlobuf, outt_hbm.at[pl.ds(dst, _ROWS)], sem3)
        ot1 = pltpu.async_copy(thibuf, outt_hbm.at[pl.ds(_B + dst, _ROWS)], sem3)
        o0.wait()
        o1.wait()
        ot0.wait()
        ot1.wait()

    @pl.when(jnp.logical_not(spec_ok))
    def _():
        a0.wait()
        a1.wait()
        t0.wait()
        t1.wait()

    @pl.when(case_a)
    def _():
        src = pl.multiple_of(src_f, _ROWS)
        pltpu.sync_copy(feats_hbm.at[pl.ds(src, _ROWS)], fbuf)
        pltpu.sync_copy(tcat_hbm.at[pl.ds(src, _ROWS)], tlobuf)
        pltpu.sync_copy(tcat_hbm.at[pl.ds(_B + src, _ROWS)], thibuf)

    @pl.when(case_b)
    def _():
        src = pl.multiple_of(g0, _ROWS)
        pltpu.sync_copy(fmem_hbm.at[pl.ds(src, _ROWS)], fbuf)
        pltpu.sync_copy(tmcat_hbm.at[pl.ds(src, _ROWS)], tlobuf)
        pltpu.sync_copy(tmcat_hbm.at[pl.ds(_K + src, _ROWS)], thibuf)

    @pl.when(case_c)
    def _():
        iota = lax.iota(jnp.int32, 16)

        def group(gi, carry):
            off = gi * _G
            c0 = g0 + off
            gvec = c0 + iota
            validv = jnp.logical_and(gvec >= ws, gvec < ws + _B)
            fidx = jnp.clip(gvec - ws, 0, _B - 1)

            pltpu.async_copy(feats_hbm.at[fidx],
                             fbuf.at[pl.ds(off, _G)], sem).wait()
            pltpu.async_copy(fmem_hbm.at[gvec], mstage, sem).wait()

            def fixrow(r, c2):
                g = c0 + r
                valid = jnp.logical_and(g >= ws, g < ws + _B)

                @pl.when(jnp.logical_not(valid))
                def _():
                    for jc in range(_D // 16):
                        fbuf[off + r, pl.ds(jc * 16, 16)] = (
                            mstage[r, pl.ds(jc * 16, 16)])

                return c2

            lax.fori_loop(jnp.int32(0), jnp.int32(_G), fixrow, jnp.int32(0))

            for pbase, mbase, pbuf in ((0, 0, tlobuf), (_B, _K, thibuf)):
                pltpu.async_copy(tcat_hbm.at[pbase + fidx],
                                 pbuf.at[pl.ds(off, _G)], sem).wait()
                pltpu.async_copy(tmcat_hbm.at[mbase + gvec], tstage,
                                 sem).wait()
                pbuf[pl.ds(off, _G)] = jnp.where(
                    validv, pbuf[pl.ds(off, _G)], tstage[...])

            return carry

        lax.fori_loop(jnp.int32(0), jnp.int32(_NGRP), group, jnp.int32(0))

    @pl.when(jnp.logical_not(spec_ok))
    def _():
        pltpu.sync_copy(fbuf, outf_hbm.at[pl.ds(dst, _ROWS)])
        pltpu.sync_copy(tlobuf, outt_hbm.at[pl.ds(dst, _ROWS)])
        pltpu.sync_copy(thibuf, outt_hbm.at[pl.ds(_B + dst, _ROWS)])


_xbm_call = functools.partial(
    pl.kernel,
    out_type=[
        jax.ShapeDtypeStruct((_B, _D), jnp.float32),
        jax.ShapeDtypeStruct((2 * _B,), jnp.int32),
    ],
    mesh=plsc.VectorSubcoreMesh(core_axis_name="c", subcore_axis_name="s"),
    compiler_params=pltpu.CompilerParams(needs_layout_passes=False),
    scratch_types=[
        pltpu.VMEM((16,), jnp.int32),
        pltpu.VMEM((_ROWS, _D), jnp.float32),
        pltpu.VMEM((_ROWS,), jnp.int32),
        pltpu.VMEM((_ROWS,), jnp.int32),
        pltpu.VMEM((_G, _D), jnp.float32),
        pltpu.VMEM((_G,), jnp.int32),
        pltpu.SemaphoreType.DMA,
        pltpu.SemaphoreType.DMA,
        pltpu.SemaphoreType.DMA,
    ],
)(_xbm_body)


def kernel(feats, targets, feats_mem, targets_mem, ptr, total_count):
    q = feats.shape[0]
    ptr32 = jnp.clip(ptr, 0, _K + 1).astype(jnp.int32)
    tc32 = jnp.clip(total_count, 0, _K + 200).astype(jnp.int32)
    wrap = ptr32 + q > _K
    write_start = jnp.where(wrap, _K - q, ptr32)
    write_start = jnp.clip(write_start, 0, _K - q)
    new_ptr = jnp.where(wrap, 0, ptr32 + q)
    is_full = tc32 + q >= _K
    out_start = jnp.where(is_full, 0, new_ptr - q)
    out_start = jnp.clip(out_start, 0, _K - q)

    params = jnp.stack([write_start, out_start]).astype(jnp.int32)
    params = jnp.pad(params, (0, 14))

    t64 = targets.reshape(q)
    t_cat = jnp.concatenate(
        [t64.astype(jnp.int32), (t64 >> 32).astype(jnp.int32)])

    def _tmem_slow(tm):
        tm = lax.optimization_barrier(tm).reshape(_K)
        return jnp.concatenate(
            [tm.astype(jnp.int32), (tm >> 32).astype(jnp.int32)])

    tm_cat = lax.cond(
        write_start == out_start,
        lambda tm: jnp.zeros((2 * _K,), jnp.int32),
        _tmem_slow,
        targets_mem)

    out_feats, out_cat = _xbm_call(params, feats, t_cat, feats_mem, tm_cat)
    out_targets = (
        (out_cat[_B:].astype(jnp.int64) << 32)
        | (out_cat[:_B].astype(jnp.int64) & jnp.int64(0xFFFFFFFF))
    ).reshape(q, 1)
    return (out_feats, out_targets)
